# Initial kernel scaffold; baseline (speedup 1.0000x reference)
import jax
import jax.numpy as jnp
from jax.experimental import pallas as pl


def kernel(x, Wq, Wk, Wv, Wo):
    raise NotImplementedError("write your pallas kernel here")


# speedup vs baseline: 1.0010x; 1.0010x over previous
import math
import functools

import jax
import jax.numpy as jnp
import numpy as np
from jax import lax
from jax.experimental import pallas as pl
from jax.experimental.pallas import tpu as pltpu

_B, _T, _D = 2, 2048, 1024
_H = 16
_R = 256


def _sim_body(xa_ref, xb_ref, sim_ref):
    xa = xa_ref[0]
    xb = xb_ref[0]
    an = xa / jnp.maximum(
        jnp.sqrt(jnp.sum(xa * xa, axis=-1, keepdims=True)), 1e-12)
    bn = xb / jnp.maximum(
        jnp.sqrt(jnp.sum(xb * xb, axis=-1, keepdims=True)), 1e-12)
    sim_ref[0] = jax.lax.dot_general(
        an, bn, dimension_numbers=(((1,), (1,)), ((), ())),
        preferred_element_type=jnp.float32)


def _sim_pallas(x_a, x_b):
    Bb, Ta, Dd = x_a.shape
    Tb = x_b.shape[1]
    return pl.pallas_call(
        _sim_body,
        grid=(Bb,),
        in_specs=[
            pl.BlockSpec((1, Ta, Dd), lambda b: (b, 0, 0)),
            pl.BlockSpec((1, Tb, Dd), lambda b: (b, 0, 0)),
        ],
        out_specs=pl.BlockSpec((1, Ta, Tb), lambda b: (b, 0, 0)),
        out_shape=jax.ShapeDtypeStruct((Bb, Ta, Tb), jnp.float32),
    )(x_a, x_b)


def _match(x, r):
    Bb, Tt, Dd = x.shape
    x_a = x[:, 0::2, :]
    x_b = x[:, 1::2, :]
    T_a = x_a.shape[1]
    T_b = x_b.shape[1]
    sim = _sim_pallas(x_a, x_b)
    batch_ar = jnp.arange(Bb)

    def step(carry, _):
        used_a, used_b = carry
        masked = jnp.where(used_a[:, :, None] | used_b[:, None, :], -jnp.inf, sim)
        flat = masked.reshape(Bb, T_a * T_b)
        idx = jnp.argmax(flat, axis=1)
        ai = idx // T_b
        bi = idx % T_b
        used_a = used_a.at[batch_ar, ai].set(True)
        used_b = used_b.at[batch_ar, bi].set(True)
        return (used_a, used_b), (ai, bi)

    init = (jnp.zeros((Bb, T_a), dtype=bool), jnp.zeros((Bb, T_b), dtype=bool))
    _, (a_seq, b_seq) = lax.scan(step, init, None, length=r)
    a_idx = a_seq.T
    b_idx = b_seq.T
    counts = jnp.ones((Bb, Tt), dtype=x.dtype)
    counts = counts.at[batch_ar[:, None], a_idx * 2].set(2.0)
    counts = counts.at[batch_ar[:, None], b_idx * 2 + 1].set(2.0)
    unmerge_weights = (1.0 / counts)[..., None]
    return a_idx, b_idx, unmerge_weights


def kernel(x, Wq, Wk, Wv, Wo):
    r = _R
    n_heads = _H
    Bb, Tt, Dd = x.shape
    r = min(r, Tt // 2)
    a_idx, b_idx, unmerge_weights = _match(lax.stop_gradient(x), r)
    x_a = x[:, 0::2, :]
    x_b = x[:, 1::2, :]
    T_a = x_a.shape[1]
    T_b = x_b.shape[1]
    batch_ar = jnp.arange(Bb)
    matched_a = jnp.take_along_axis(x_a, a_idx[:, :, None], axis=1)
    matched_b = jnp.take_along_axis(x_b, b_idx[:, :, None], axis=1)
    merged = (matched_a + matched_b) * 0.5
    x_a_mod = x_a.at[batch_ar[:, None], a_idx].set(merged)
    keep = jnp.ones((Bb, T_b), dtype=bool).at[batch_ar[:, None], b_idx].set(False)
    kept_idx = jnp.argsort(jnp.where(keep, 0, 1), axis=1)[:, :T_b - r]
    unmatched_b = jnp.take_along_axis(x_b, kept_idx[:, :, None], axis=1)
    merged_x = jnp.concatenate([x_a_mod, unmatched_b], axis=1)
    Bm, Tm, _ = merged_x.shape
    hd = Dd // n_heads
    q = (merged_x @ Wq.T).reshape(Bm, Tm, n_heads, hd).transpose(0, 2, 1, 3)
    k = (merged_x @ Wk.T).reshape(Bm, Tm, n_heads, hd).transpose(0, 2, 1, 3)
    v = (merged_x @ Wv.T).reshape(Bm, Tm, n_heads, hd).transpose(0, 2, 1, 3)
    attn = jax.nn.softmax((q @ k.transpose(0, 1, 3, 2)) / math.sqrt(hd), axis=-1)
    out = (attn @ v).transpose(0, 2, 1, 3).reshape(Bm, Tm, Dd)
    attn_out = out @ Wo.T
    x_a_out = attn_out[:, :T_a, :]
    un_b = attn_out[:, T_a:, :]
    x_b_res = jnp.zeros((Bb, T_b, Dd), dtype=attn_out.dtype)
    x_b_res = x_b_res.at[batch_ar[:, None], kept_idx].set(un_b)
    merged_vals = jnp.take_along_axis(x_a_out, a_idx[:, :, None], axis=1)
    x_b_res = x_b_res.at[batch_ar[:, None], b_idx].set(merged_vals)
    output = jnp.zeros((Bb, Tt, Dd), dtype=attn_out.dtype)
    output = output.at[:, 0::2, :].set(x_a_out)
    output = output.at[:, 1::2, :].set(x_b_res)
    return output


# speedup vs baseline: 12.5868x; 12.5748x over previous
import math
import functools

import jax
import jax.numpy as jnp
import numpy as np
from jax import lax
from jax.experimental import pallas as pl
from jax.experimental.pallas import tpu as pltpu

_B, _T, _D = 2, 2048, 1024
_H = 16
_R = 256
_TA = _T // 2
_NEG_INF = float("-inf")


def _match_body(xa_ref, xb_ref, merged_ref, tmask_ref, srcodd_ref, sim_ref):
    xa = xa_ref[0]
    xb = xb_ref[0]
    an = xa / jnp.maximum(
        jnp.sqrt(jnp.sum(xa * xa, axis=-1, keepdims=True)), 1e-12)
    bn = xb / jnp.maximum(
        jnp.sqrt(jnp.sum(xb * xb, axis=-1, keepdims=True)), 1e-12)
    sim_ref[...] = jax.lax.dot_general(
        an, bn, dimension_numbers=(((1,), (1,)), ((), ())),
        preferred_element_type=jnp.float32)
    sim_t = jax.lax.dot_general(
        bn, an, dimension_numbers=(((1,), (1,)), ((), ())),
        preferred_element_type=jnp.float32)
    rmax0 = jnp.max(sim_t, axis=0, keepdims=True)

    merged_ref[0, pl.ds(0, _TA), :] = xa
    merged_ref[0, pl.ds(_TA, _TA), :] = xb

    lane = lax.broadcasted_iota(jnp.int32, (1, _TA), 1)
    big = jnp.int32(1 << 30)

    def cond(carry):
        step, _, _, _ = carry
        return step < _R

    def body(carry):
        step, rmax, used_b, src_odd = carry
        m = jnp.max(rmax)
        a_star = jnp.min(jnp.where(rmax == m, lane, big))
        row = sim_ref[pl.ds(a_star, 1), :]
        bv = jnp.where(used_b > 0, _NEG_INF, row)
        nm = jnp.max(bv)
        b_star = jnp.min(jnp.where(bv == nm, lane, big))
        accept = nm == m
        acc_f = accept.astype(jnp.float32)
        acc_i = accept.astype(jnp.int32)
        ra = merged_ref[0, pl.ds(a_star, 1), :]
        rb = xb_ref[0, pl.ds(b_star, 1), :]
        newrow = (ra + rb) * 0.5
        merged_ref[0, pl.ds(a_star, 1), :] = jnp.where(accept, newrow, ra)
        rmax = jnp.where(lane == a_star, jnp.where(accept, _NEG_INF, nm), rmax)
        used_b = jnp.where((lane == b_star) & accept, 1.0, used_b)
        src_odd = jnp.where((lane == b_star) & accept, a_star, src_odd)
        return step + acc_i, rmax, used_b, src_odd

    init = (jnp.int32(0), rmax0, jnp.zeros((1, _TA), jnp.float32),
            lane + jnp.int32(_TA))
    _, _, used_b, src_odd = lax.while_loop(cond, body, init)

    tmask_ref[0, :, pl.ds(0, _TA)] = jnp.zeros((1, _TA), jnp.float32)
    tmask_ref[0, :, pl.ds(_TA, _TA)] = jnp.where(used_b > 0, _NEG_INF, 0.0)
    srcodd_ref[0, :, :] = src_odd


def _match_pallas(x_a, x_b):
    Bb = x_a.shape[0]
    return pl.pallas_call(
        _match_body,
        grid=(Bb,),
        in_specs=[
            pl.BlockSpec((1, _TA, _D), lambda b: (b, 0, 0)),
            pl.BlockSpec((1, _TA, _D), lambda b: (b, 0, 0)),
        ],
        out_specs=[
            pl.BlockSpec((1, _T, _D), lambda b: (b, 0, 0)),
            pl.BlockSpec((1, 1, _T), lambda b: (b, 0, 0)),
            pl.BlockSpec((1, 1, _TA), lambda b: (b, 0, 0)),
        ],
        out_shape=[
            jax.ShapeDtypeStruct((Bb, _T, _D), jnp.float32),
            jax.ShapeDtypeStruct((Bb, 1, _T), jnp.float32),
            jax.ShapeDtypeStruct((Bb, 1, _TA), jnp.int32),
        ],
        scratch_shapes=[pltpu.VMEM((_TA, _TA), jnp.float32)],
    )(x_a, x_b)


def kernel(x, Wq, Wk, Wv, Wo):
    Bb, Tt, Dd = x.shape
    n_heads = _H
    x_a = x[:, 0::2, :]
    x_b = x[:, 1::2, :]
    merged_full, tmask, src_odd = _match_pallas(x_a, x_b)
    hd = Dd // n_heads
    Tm = Tt
    q = (merged_full @ Wq.T).reshape(Bb, Tm, n_heads, hd).transpose(0, 2, 1, 3)
    k = (merged_full @ Wk.T).reshape(Bb, Tm, n_heads, hd).transpose(0, 2, 1, 3)
    v = (merged_full @ Wv.T).reshape(Bb, Tm, n_heads, hd).transpose(0, 2, 1, 3)
    scores = (q @ k.transpose(0, 1, 3, 2)) / math.sqrt(hd)
    scores = scores + tmask[:, :, None, :]
    attn = jax.nn.softmax(scores, axis=-1)
    out = (attn @ v).transpose(0, 2, 1, 3).reshape(Bb, Tm, Dd)
    attn_out = out @ Wo.T
    ar = jnp.broadcast_to(jnp.arange(_TA, dtype=jnp.int32)[None], (Bb, _TA))
    src_full = jnp.stack([ar, src_odd[:, 0, :]], axis=-1).reshape(Bb, Tt)
    return jnp.take_along_axis(attn_out, src_full[:, :, None], axis=1)


# speedup vs baseline: 12.7437x; 1.0125x over previous
import math
import functools

import jax
import jax.numpy as jnp
import numpy as np
from jax import lax
from jax.experimental import pallas as pl
from jax.experimental.pallas import tpu as pltpu
from jax.experimental.pallas import tpu_sc as plsc

_B, _T, _D = 2, 2048, 1024
_H = 16
_R = 256
_TA = _T // 2
_HD = _D // _H
_NEG_INF = float("-inf")


def _match_body(xa_ref, xb_ref, merged_ref, tmask_ref, srcodd_ref, sim_ref):
    xa = xa_ref[0]
    xb = xb_ref[0]
    an = xa / jnp.maximum(
        jnp.sqrt(jnp.sum(xa * xa, axis=-1, keepdims=True)), 1e-12)
    bn = xb / jnp.maximum(
        jnp.sqrt(jnp.sum(xb * xb, axis=-1, keepdims=True)), 1e-12)
    sim_ref[...] = jax.lax.dot_general(
        an, bn, dimension_numbers=(((1,), (1,)), ((), ())),
        preferred_element_type=jnp.float32)
    sim_t = jax.lax.dot_general(
        bn, an, dimension_numbers=(((1,), (1,)), ((), ())),
        preferred_element_type=jnp.float32)
    rmax0 = jnp.max(sim_t, axis=0, keepdims=True)

    merged_ref[0, pl.ds(0, _TA), :] = xa
    merged_ref[0, pl.ds(_TA, _TA), :] = xb

    lane = lax.broadcasted_iota(jnp.int32, (1, _TA), 1)
    big = jnp.int32(1 << 30)

    def cond(carry):
        step, _, _, _ = carry
        return step < _R

    def body(carry):
        step, rmax, used_b, src_odd = carry
        m = jnp.max(rmax)
        a_star = jnp.min(jnp.where(rmax == m, lane, big))
        row = sim_ref[pl.ds(a_star, 1), :]
        bv = jnp.where(used_b > 0, _NEG_INF, row)
        nm = jnp.max(bv)
        b_star = jnp.min(jnp.where(bv == nm, lane, big))
        accept = nm == m
        acc_i = accept.astype(jnp.int32)
        ra = merged_ref[0, pl.ds(a_star, 1), :]
        rb = xb_ref[0, pl.ds(b_star, 1), :]
        newrow = (ra + rb) * 0.5
        merged_ref[0, pl.ds(a_star, 1), :] = jnp.where(accept, newrow, ra)
        rmax = jnp.where(lane == a_star, jnp.where(accept, _NEG_INF, nm), rmax)
        used_b = jnp.where((lane == b_star) & accept, 1.0, used_b)
        src_odd = jnp.where((lane == b_star) & accept, a_star, src_odd)
        return step + acc_i, rmax, used_b, src_odd

    init = (jnp.int32(0), rmax0, jnp.zeros((1, _TA), jnp.float32),
            lane + jnp.int32(_TA))
    _, _, used_b, src_odd = lax.while_loop(cond, body, init)

    tmask_ref[0, :, pl.ds(0, _TA)] = jnp.zeros((1, _TA), jnp.float32)
    tmask_ref[0, :, pl.ds(_TA, _TA)] = jnp.where(used_b > 0, _NEG_INF, 0.0)
    srcodd_ref[0, :, :] = src_odd


def _match_pallas(x_a, x_b):
    Bb = x_a.shape[0]
    return pl.pallas_call(
        _match_body,
        grid=(Bb,),
        in_specs=[
            pl.BlockSpec((1, _TA, _D), lambda b: (b, 0, 0)),
            pl.BlockSpec((1, _TA, _D), lambda b: (b, 0, 0)),
        ],
        out_specs=[
            pl.BlockSpec((1, _T, _D), lambda b: (b, 0, 0)),
            pl.BlockSpec((1, 1, _T), lambda b: (b, 0, 0)),
            pl.BlockSpec((1, 1, _TA), lambda b: (b, 0, 0)),
        ],
        out_shape=[
            jax.ShapeDtypeStruct((Bb, _T, _D), jnp.float32),
            jax.ShapeDtypeStruct((Bb, 1, _T), jnp.float32),
            jax.ShapeDtypeStruct((Bb, 1, _TA), jnp.int32),
        ],
        scratch_shapes=[pltpu.VMEM((_TA, _TA), jnp.float32)],
    )(x_a, x_b)


_QT = 512


def _attn_body(m_ref, tmask_ref, wq_ref, wk_ref, wv_ref, wo_ref, out_ref):
    h = pl.program_id(1)
    m = m_ref[0]
    dims = (((1,), (1,)), ((), ()))
    q = jax.lax.dot_general(m, wq_ref[...], dims,
                            preferred_element_type=jnp.float32)
    k = jax.lax.dot_general(m, wk_ref[...], dims,
                            preferred_element_type=jnp.float32)
    v = jax.lax.dot_general(m, wv_ref[...], dims,
                            preferred_element_type=jnp.float32)
    mask = tmask_ref[0]
    scale = 1.0 / math.sqrt(_HD)
    for i in range(_T // _QT):
        qi = q[i * _QT:(i + 1) * _QT]
        s = jax.lax.dot_general(qi, k, dims,
                                preferred_element_type=jnp.float32)
        s = s * scale + mask
        s = s - jnp.max(s, axis=-1, keepdims=True)
        p = jnp.exp(s)
        p = p / jnp.sum(p, axis=-1, keepdims=True)
        o = jax.lax.dot_general(p, v, (((1,), (0,)), ((), ())),
                                preferred_element_type=jnp.float32)
        part = jax.lax.dot_general(o, wo_ref[...], (((1,), (0,)), ((), ())),
                                   preferred_element_type=jnp.float32)

        @pl.when(h == 0)
        def _():
            out_ref[0, i * _QT:(i + 1) * _QT, :] = part

        @pl.when(h > 0)
        def _():
            out_ref[0, i * _QT:(i + 1) * _QT, :] += part


def _attn_pallas(merged, tmask, Wq, Wk, Wv, Wo):
    Bb = merged.shape[0]
    return pl.pallas_call(
        _attn_body,
        grid=(Bb, _H),
        in_specs=[
            pl.BlockSpec((1, _T, _D), lambda b, h: (b, 0, 0)),
            pl.BlockSpec((1, 1, _T), lambda b, h: (b, 0, 0)),
            pl.BlockSpec((_HD, _D), lambda b, h: (h, 0)),
            pl.BlockSpec((_HD, _D), lambda b, h: (h, 0)),
            pl.BlockSpec((_HD, _D), lambda b, h: (h, 0)),
            pl.BlockSpec((_HD, _D), lambda b, h: (h, 0)),
        ],
        out_specs=pl.BlockSpec((1, _T, _D), lambda b, h: (b, 0, 0)),
        out_shape=jax.ShapeDtypeStruct((Bb, _T, _D), jnp.float32),
    )(merged, tmask, Wq, Wk, Wv, Wo.T)


_SC_CHUNK = 64


def _unmerge_sc(attn2, srcodd2):
    info = plsc.get_sparse_core_info()
    nc, ns = info.num_cores, info.num_subcores
    nw = nc * ns
    mesh = plsc.VectorSubcoreMesh(core_axis_name="c", subcore_axis_name="s")
    n_rows = attn2.shape[0]
    t_per_w = (_B * _TA) // nw

    @functools.partial(
        pl.kernel, mesh=mesh,
        out_type=jax.ShapeDtypeStruct((n_rows, _D), jnp.float32),
        scratch_types=[
            pltpu.VMEM((_SC_CHUNK,), jnp.int32),
            pltpu.VMEM((_SC_CHUNK,), jnp.int32),
            pltpu.VMEM((_SC_CHUNK, _D), jnp.float32),
            pltpu.SemaphoreType.DMA,
        ],
    )
    def k(attn_hbm, srcodd_hbm, out_hbm, idx_v, dest_v, rows_v, sem):
        wid = lax.axis_index("s") * nc + lax.axis_index("c")
        t0 = wid * t_per_w
        b = t0 // _TA
        bt0 = t0 - b * _TA
        iota = lax.iota(jnp.int32, 16)
        for c0 in range(0, t_per_w, _SC_CHUNK):
            pltpu.sync_copy(attn_hbm.at[pl.ds(b * _T + bt0 + c0, _SC_CHUNK)],
                            rows_v)
            for j in range(_SC_CHUNK // 16):
                dest_v[pl.ds(j * 16, 16)] = (
                    b * _T + 2 * (bt0 + c0 + j * 16 + iota))
            pltpu.async_copy(rows_v, out_hbm.at[dest_v], sem).wait()
            pltpu.sync_copy(srcodd_hbm.at[pl.ds(t0 + c0, _SC_CHUNK)], idx_v)
            for j in range(_SC_CHUNK // 16):
                idx_v[pl.ds(j * 16, 16)] = (
                    idx_v[pl.ds(j * 16, 16)] + b * _T)
                dest_v[pl.ds(j * 16, 16)] = (
                    b * _T + 2 * (bt0 + c0 + j * 16 + iota) + 1)
            pltpu.async_copy(attn_hbm.at[idx_v], rows_v, sem).wait()
            pltpu.async_copy(rows_v, out_hbm.at[dest_v], sem).wait()

    return k(attn2, srcodd2)


def kernel(x, Wq, Wk, Wv, Wo):
    Bb, Tt, Dd = x.shape
    x_a = x[:, 0::2, :]
    x_b = x[:, 1::2, :]
    merged_full, tmask, src_odd = _match_pallas(x_a, x_b)
    attn_out = _attn_pallas(merged_full, tmask, Wq, Wk, Wv, Wo)
    out2 = _unmerge_sc(attn_out.reshape(Bb * Tt, Dd),
                       src_odd.reshape(Bb * _TA))
    return out2.reshape(Bb, Tt, Dd)


# speedup vs baseline: 13.0411x; 1.0233x over previous
import math
import functools

import jax
import jax.numpy as jnp
import numpy as np
from jax import lax
from jax.experimental import pallas as pl
from jax.experimental.pallas import tpu as pltpu
from jax.experimental.pallas import tpu_sc as plsc

_B, _T, _D = 2, 2048, 1024
_H = 16
_R = 256
_TA = _T // 2
_HD = _D // _H
_NEG_INF = float("-inf")


def _match_body(xa_ref, xb_ref, merged_ref, tmask_ref, srcodd_ref, sim_ref):
    lane = lax.broadcasted_iota(jnp.int32, (1, _TA), 1)
    big = jnp.int32(1 << 30)

    rmax0 = []
    for b in range(_B):
        xa = xa_ref[b]
        xb = xb_ref[b]
        an = xa / jnp.maximum(
            jnp.sqrt(jnp.sum(xa * xa, axis=-1, keepdims=True)), 1e-12)
        bn = xb / jnp.maximum(
            jnp.sqrt(jnp.sum(xb * xb, axis=-1, keepdims=True)), 1e-12)
        sim_ref[b] = jax.lax.dot_general(
            an, bn, dimension_numbers=(((1,), (1,)), ((), ())),
            preferred_element_type=jnp.float32)
        sim_t = jax.lax.dot_general(
            bn, an, dimension_numbers=(((1,), (1,)), ((), ())),
            preferred_element_type=jnp.float32)
        rmax0.append(jnp.max(sim_t, axis=0, keepdims=True))
        merged_ref[b, pl.ds(0, _TA), :] = xa
        merged_ref[b, pl.ds(_TA, _TA), :] = xb

    def one_batch(bi, step, rmax, used_b, src_odd):
        m = jnp.max(rmax)
        a_star = jnp.min(jnp.where(rmax == m, lane, big))
        row = sim_ref[bi, pl.ds(a_star, 1), :]
        bv = jnp.where(used_b > 0, _NEG_INF, row)
        nm = jnp.max(bv)
        b_star = jnp.min(jnp.where(bv == nm, lane, big))
        accept = (nm == m) & (step < _R)
        ra = merged_ref[bi, pl.ds(a_star, 1), :]
        rb = xb_ref[bi, pl.ds(b_star, 1), :]
        newrow = (ra + rb) * 0.5
        merged_ref[bi, pl.ds(a_star, 1), :] = jnp.where(accept, newrow, ra)
        rmax = jnp.where(lane == a_star, jnp.where(accept, _NEG_INF, nm), rmax)
        used_b = jnp.where((lane == b_star) & accept, 1.0, used_b)
        src_odd = jnp.where((lane == b_star) & accept, a_star, src_odd)
        return step + accept.astype(jnp.int32), rmax, used_b, src_odd

    def cond(carry):
        return (carry[0][0] < _R) | (carry[1][0] < _R)

    def body(carry):
        return tuple(one_batch(bi, *carry[bi]) for bi in range(_B))

    zero_f = jnp.zeros((1, _TA), jnp.float32)
    init = tuple((jnp.int32(0), rmax0[b], zero_f, lane + jnp.int32(_TA))
                 for b in range(_B))
    final = lax.while_loop(cond, body, init)

    for b in range(_B):
        _, _, used_b, src_odd = final[b]
        tmask_ref[b, :, pl.ds(0, _TA)] = jnp.zeros((1, _TA), jnp.float32)
        tmask_ref[b, :, pl.ds(_TA, _TA)] = jnp.where(used_b > 0, _NEG_INF, 0.0)
        srcodd_ref[b, :, :] = src_odd


def _match_pallas(x_a, x_b):
    Bb = x_a.shape[0]
    return pl.pallas_call(
        _match_body,
        in_specs=[
            pl.BlockSpec((Bb, _TA, _D), lambda: (0, 0, 0)),
            pl.BlockSpec((Bb, _TA, _D), lambda: (0, 0, 0)),
        ],
        out_specs=[
            pl.BlockSpec((Bb, _T, _D), lambda: (0, 0, 0)),
            pl.BlockSpec((Bb, 1, _T), lambda: (0, 0, 0)),
            pl.BlockSpec((Bb, 1, _TA), lambda: (0, 0, 0)),
        ],
        out_shape=[
            jax.ShapeDtypeStruct((Bb, _T, _D), jnp.float32),
            jax.ShapeDtypeStruct((Bb, 1, _T), jnp.float32),
            jax.ShapeDtypeStruct((Bb, 1, _TA), jnp.int32),
        ],
        scratch_shapes=[pltpu.VMEM((Bb, _TA, _TA), jnp.float32)],
    )(x_a, x_b)


_QT = 512


def _attn_body(m_ref, tmask_ref, wq_ref, wk_ref, wv_ref, wo_ref, out_ref):
    h = pl.program_id(1)
    m = m_ref[0].astype(jnp.bfloat16)
    dims = (((1,), (1,)), ((), ()))
    q = jax.lax.dot_general(m, wq_ref[...].astype(jnp.bfloat16), dims,
                            preferred_element_type=jnp.float32)
    k = jax.lax.dot_general(m, wk_ref[...].astype(jnp.bfloat16), dims,
                            preferred_element_type=jnp.float32)
    v = jax.lax.dot_general(m, wv_ref[...].astype(jnp.bfloat16), dims,
                            preferred_element_type=jnp.float32
                            ).astype(jnp.bfloat16)
    mask = tmask_ref[0]
    scale = 1.0 / math.sqrt(_HD)
    kb = k.astype(jnp.bfloat16)
    qb = q.astype(jnp.bfloat16)
    for i in range(_T // _QT):
        qi = qb[i * _QT:(i + 1) * _QT]
        s = jax.lax.dot_general(qi, kb, dims,
                                preferred_element_type=jnp.float32)
        s = s * scale + mask
        s = s - jnp.max(s, axis=-1, keepdims=True)
        p = jnp.exp(s)
        p = p / jnp.sum(p, axis=-1, keepdims=True)
        o = jax.lax.dot_general(p.astype(jnp.bfloat16), v,
                                (((1,), (0,)), ((), ())),
                                preferred_element_type=jnp.float32
                                ).astype(jnp.bfloat16)
        part = jax.lax.dot_general(o, wo_ref[...].astype(jnp.bfloat16),
                                   (((1,), (0,)), ((), ())),
                                   preferred_element_type=jnp.float32)

        @pl.when(h == 0)
        def _():
            out_ref[0, i * _QT:(i + 1) * _QT, :] = part

        @pl.when(h > 0)
        def _():
            out_ref[0, i * _QT:(i + 1) * _QT, :] += part


def _attn_pallas(merged, tmask, Wq, Wk, Wv, Wo):
    Bb = merged.shape[0]
    return pl.pallas_call(
        _attn_body,
        grid=(Bb, _H),
        in_specs=[
            pl.BlockSpec((1, _T, _D), lambda b, h: (b, 0, 0)),
            pl.BlockSpec((1, 1, _T), lambda b, h: (b, 0, 0)),
            pl.BlockSpec((_HD, _D), lambda b, h: (h, 0)),
            pl.BlockSpec((_HD, _D), lambda b, h: (h, 0)),
            pl.BlockSpec((_HD, _D), lambda b, h: (h, 0)),
            pl.BlockSpec((_HD, _D), lambda b, h: (h, 0)),
        ],
        out_specs=pl.BlockSpec((1, _T, _D), lambda b, h: (b, 0, 0)),
        out_shape=jax.ShapeDtypeStruct((Bb, _T, _D), jnp.float32),
    )(merged, tmask, Wq, Wk, Wv, Wo.T)


_SC_CHUNK = 64


def _unmerge_sc(attn2, srcodd2):
    info = plsc.get_sparse_core_info()
    nc, ns = info.num_cores, info.num_subcores
    nw = nc * ns
    mesh = plsc.VectorSubcoreMesh(core_axis_name="c", subcore_axis_name="s")
    n_rows = attn2.shape[0]
    t_per_w = (_B * _TA) // nw

    @functools.partial(
        pl.kernel, mesh=mesh,
        out_type=jax.ShapeDtypeStruct((n_rows, _D), jnp.float32),
        scratch_types=[
            pltpu.VMEM((_SC_CHUNK,), jnp.int32),
            pltpu.VMEM((_SC_CHUNK,), jnp.int32),
            pltpu.VMEM((_SC_CHUNK, _D), jnp.float32),
            pltpu.SemaphoreType.DMA,
        ],
    )
    def k(attn_hbm, srcodd_hbm, out_hbm, idx_v, dest_v, rows_v, sem):
        wid = lax.axis_index("s") * nc + lax.axis_index("c")
        t0 = wid * t_per_w
        b = t0 // _TA
        bt0 = t0 - b * _TA
        iota = lax.iota(jnp.int32, 16)
        for c0 in range(0, t_per_w, _SC_CHUNK):
            pltpu.sync_copy(attn_hbm.at[pl.ds(b * _T + bt0 + c0, _SC_CHUNK)],
                            rows_v)
            for j in range(_SC_CHUNK // 16):
                dest_v[pl.ds(j * 16, 16)] = (
                    b * _T + 2 * (bt0 + c0 + j * 16 + iota))
            pltpu.async_copy(rows_v, out_hbm.at[dest_v], sem).wait()
            pltpu.sync_copy(srcodd_hbm.at[pl.ds(t0 + c0, _SC_CHUNK)], idx_v)
            for j in range(_SC_CHUNK // 16):
                idx_v[pl.ds(j * 16, 16)] = (
                    idx_v[pl.ds(j * 16, 16)] + b * _T)
                dest_v[pl.ds(j * 16, 16)] = (
                    b * _T + 2 * (bt0 + c0 + j * 16 + iota) + 1)
            pltpu.async_copy(attn_hbm.at[idx_v], rows_v, sem).wait()
            pltpu.async_copy(rows_v, out_hbm.at[dest_v], sem).wait()

    return k(attn2, srcodd2)


def kernel(x, Wq, Wk, Wv, Wo):
    Bb, Tt, Dd = x.shape
    x_a = x[:, 0::2, :]
    x_b = x[:, 1::2, :]
    merged_full, tmask, src_odd = _match_pallas(x_a, x_b)
    attn_out = _attn_pallas(merged_full, tmask, Wq, Wk, Wv, Wo)
    out2 = _unmerge_sc(attn_out.reshape(Bb * Tt, Dd),
                       src_odd.reshape(Bb * _TA))
    return out2.reshape(Bb, Tt, Dd)


# speedup vs baseline: 14.9920x; 1.1496x over previous
import math
import functools

import jax
import jax.numpy as jnp
import numpy as np
from jax import lax
from jax.experimental import pallas as pl
from jax.experimental.pallas import tpu as pltpu
from jax.experimental.pallas import tpu_sc as plsc

_B, _T, _D = 2, 2048, 1024
_H = 16
_R = 256
_TA = _T // 2
_HD = _D // _H
_NEG_INF = float("-inf")


def _match_body(xa_ref, xb_ref, merged_ref, tmask_ref, srcodd_ref, sim_ref,
                rmax_ref, used_ref):
    lane = lax.broadcasted_iota(jnp.int32, (1, _TA), 1)
    big = jnp.int32(1 << 30)

    for b in range(_B):
        xa = xa_ref[b]
        xb = xb_ref[b]
        an = xa / jnp.maximum(
            jnp.sqrt(jnp.sum(xa * xa, axis=-1, keepdims=True)), 1e-12)
        bn = xb / jnp.maximum(
            jnp.sqrt(jnp.sum(xb * xb, axis=-1, keepdims=True)), 1e-12)
        sim_ref[b] = jax.lax.dot_general(
            an, bn, dimension_numbers=(((1,), (1,)), ((), ())),
            preferred_element_type=jnp.float32)
        sim_t = jax.lax.dot_general(
            bn, an, dimension_numbers=(((1,), (1,)), ((), ())),
            preferred_element_type=jnp.float32)
        rmax_ref[b] = jnp.max(sim_t, axis=0, keepdims=True)
        used_ref[b] = jnp.zeros((1, _TA), jnp.float32)
        srcodd_ref[b] = lane + jnp.int32(_TA)
        merged_ref[b, pl.ds(0, _TA), :] = xa
        merged_ref[b, pl.ds(_TA, _TA), :] = xb

    def one_batch(bi, step):
        rmax = rmax_ref[bi]
        m = jnp.max(rmax)
        a_star = jnp.min(jnp.where(rmax == m, lane, big))
        row = sim_ref[bi, pl.ds(a_star, 1), :]
        used_b = used_ref[bi]
        bv = jnp.where(used_b > 0, _NEG_INF, row)
        nm = jnp.max(bv)
        b_star = jnp.min(jnp.where(bv == nm, lane, big))
        accept = (nm == m) & (step < _R)
        ra = merged_ref[bi, pl.ds(a_star, 1), :]
        rb = xb_ref[bi, pl.ds(b_star, 1), :]
        merged_ref[bi, pl.ds(a_star, 1), :] = jnp.where(
            accept, (ra + rb) * 0.5, ra)
        rmax_ref[bi] = jnp.where(
            lane == a_star, jnp.where(accept, _NEG_INF, nm), rmax)
        used_ref[bi] = jnp.where((lane == b_star) & accept, 1.0, used_b)
        srcodd_ref[bi] = jnp.where(
            (lane == b_star) & accept, a_star, srcodd_ref[bi])
        return step + accept.astype(jnp.int32)

    def cond(carry):
        return (carry[0] < _R) | (carry[1] < _R)

    def body(carry):
        return tuple(one_batch(bi, carry[bi]) for bi in range(_B))

    lax.while_loop(cond, body, (jnp.int32(0), jnp.int32(0)))

    for b in range(_B):
        used_b = used_ref[b]
        tmask_ref[b, :, pl.ds(0, _TA)] = jnp.zeros((1, _TA), jnp.float32)
        tmask_ref[b, :, pl.ds(_TA, _TA)] = jnp.where(used_b > 0, _NEG_INF, 0.0)


def _match_pallas(x_a, x_b):
    Bb = x_a.shape[0]
    return pl.pallas_call(
        _match_body,
        in_specs=[
            pl.BlockSpec((Bb, _TA, _D), lambda: (0, 0, 0)),
            pl.BlockSpec((Bb, _TA, _D), lambda: (0, 0, 0)),
        ],
        out_specs=[
            pl.BlockSpec((Bb, _T, _D), lambda: (0, 0, 0)),
            pl.BlockSpec((Bb, 1, _T), lambda: (0, 0, 0)),
            pl.BlockSpec((Bb, 1, _TA), lambda: (0, 0, 0)),
        ],
        out_shape=[
            jax.ShapeDtypeStruct((Bb, _T, _D), jnp.float32),
            jax.ShapeDtypeStruct((Bb, 1, _T), jnp.float32),
            jax.ShapeDtypeStruct((Bb, 1, _TA), jnp.int32),
        ],
        scratch_shapes=[
            pltpu.VMEM((Bb, _TA, _TA), jnp.float32),
            pltpu.VMEM((Bb, 1, _TA), jnp.float32),
            pltpu.VMEM((Bb, 1, _TA), jnp.float32),
        ],
    )(x_a, x_b)


def _qkv_body(m_ref, w_ref, out_ref):
    m = m_ref[0].astype(jnp.bfloat16)
    w = w_ref[0]
    qkv = jax.lax.dot_general(m, w, (((1,), (1,)), ((), ())),
                              preferred_element_type=jnp.float32)
    out_ref[0, :, :] = qkv.astype(jnp.bfloat16)


def _qkv_pallas(merged, w_cat):
    Bb = merged.shape[0]
    return pl.pallas_call(
        _qkv_body,
        grid=(Bb, 3),
        in_specs=[
            pl.BlockSpec((1, _T, _D), lambda b, j: (b, 0, 0)),
            pl.BlockSpec((1, _D, _D), lambda b, j: (j, 0, 0)),
        ],
        out_specs=pl.BlockSpec((1, _T, _D), lambda b, j: (b, 0, j)),
        out_shape=jax.ShapeDtypeStruct((Bb, _T, 3 * _D), jnp.bfloat16),
    )(merged, w_cat)


_QT = 512


def _attn_body(q_ref, k_ref, v_ref, tmask_ref, out_ref):
    q2 = q_ref[0, 0]
    k2 = k_ref[0, 0]
    v2 = v_ref[0, 0]
    mask = tmask_ref[0]
    scale = 1.0 / math.sqrt(_HD)
    dims = (((1,), (1,)), ((), ()))
    for i in range(_T // _QT):
        qi = q2[i * _QT:(i + 1) * _QT]
        s = jax.lax.dot_general(qi, k2, dims,
                                preferred_element_type=jnp.float32)
        s = s * scale + mask
        s = s - jnp.max(s, axis=-1, keepdims=True)
        p = jnp.exp(s)
        denom = jnp.sum(p, axis=-1, keepdims=True)
        o = jax.lax.dot_general(p.astype(jnp.bfloat16), v2,
                                (((1,), (0,)), ((), ())),
                                preferred_element_type=jnp.float32)
        o = o * (1.0 / denom)
        out_ref[0, 0, i * _QT:(i + 1) * _QT, :] = o.astype(jnp.bfloat16)


def _attn_pallas(qh, kh, vh, tmask):
    Bb = qh.shape[0]
    return pl.pallas_call(
        _attn_body,
        grid=(Bb, _H),
        in_specs=[
            pl.BlockSpec((1, 1, _T, _HD), lambda b, h: (b, h, 0, 0)),
            pl.BlockSpec((1, 1, _T, _HD), lambda b, h: (b, h, 0, 0)),
            pl.BlockSpec((1, 1, _T, _HD), lambda b, h: (b, h, 0, 0)),
            pl.BlockSpec((1, 1, _T), lambda b, h: (b, 0, 0)),
        ],
        out_specs=pl.BlockSpec((1, 1, _T, _HD), lambda b, h: (b, h, 0, 0)),
        out_shape=jax.ShapeDtypeStruct((Bb, _H, _T, _HD), jnp.bfloat16),
    )(qh, kh, vh, tmask)


def _oproj_body(o_ref, w_ref, out_ref):
    o = o_ref[0]
    w = w_ref[...].astype(jnp.bfloat16)
    out_ref[0] = jax.lax.dot_general(o, w, (((1,), (1,)), ((), ())),
                                     preferred_element_type=jnp.float32)


def _oproj_pallas(o_cat, Wo):
    Bb = o_cat.shape[0]
    return pl.pallas_call(
        _oproj_body,
        grid=(Bb,),
        in_specs=[
            pl.BlockSpec((1, _T, _D), lambda b: (b, 0, 0)),
            pl.BlockSpec((_D, _D), lambda b: (0, 0)),
        ],
        out_specs=pl.BlockSpec((1, _T, _D), lambda b: (b, 0, 0)),
        out_shape=jax.ShapeDtypeStruct((Bb, _T, _D), jnp.float32),
    )(o_cat, Wo)


_SC_CHUNK = 64


def _unmerge_sc(attn2, srcodd2):
    info = plsc.get_sparse_core_info()
    nc, ns = info.num_cores, info.num_subcores
    nw = nc * ns
    mesh = plsc.VectorSubcoreMesh(core_axis_name="c", subcore_axis_name="s")
    n_rows = attn2.shape[0]
    t_per_w = (_B * _TA) // nw

    @functools.partial(
        pl.kernel, mesh=mesh,
        out_type=jax.ShapeDtypeStruct((n_rows, _D), jnp.float32),
        scratch_types=[
            pltpu.VMEM((_SC_CHUNK,), jnp.int32),
            pltpu.VMEM((_SC_CHUNK,), jnp.int32),
            pltpu.VMEM((_SC_CHUNK, _D), jnp.float32),
            pltpu.SemaphoreType.DMA,
        ],
    )
    def k(attn_hbm, srcodd_hbm, out_hbm, idx_v, dest_v, rows_v, sem):
        wid = lax.axis_index("s") * nc + lax.axis_index("c")
        t0 = wid * t_per_w
        b = t0 // _TA
        bt0 = t0 - b * _TA
        iota = lax.iota(jnp.int32, 16)
        for c0 in range(0, t_per_w, _SC_CHUNK):
            pltpu.sync_copy(attn_hbm.at[pl.ds(b * _T + bt0 + c0, _SC_CHUNK)],
                            rows_v)
            for j in range(_SC_CHUNK // 16):
                dest_v[pl.ds(j * 16, 16)] = (
                    b * _T + 2 * (bt0 + c0 + j * 16 + iota))
            pltpu.async_copy(rows_v, out_hbm.at[dest_v], sem).wait()
            pltpu.sync_copy(srcodd_hbm.at[pl.ds(t0 + c0, _SC_CHUNK)], idx_v)
            for j in range(_SC_CHUNK // 16):
                idx_v[pl.ds(j * 16, 16)] = (
                    idx_v[pl.ds(j * 16, 16)] + b * _T)
                dest_v[pl.ds(j * 16, 16)] = (
                    b * _T + 2 * (bt0 + c0 + j * 16 + iota) + 1)
            pltpu.async_copy(attn_hbm.at[idx_v], rows_v, sem).wait()
            pltpu.async_copy(rows_v, out_hbm.at[dest_v], sem).wait()

    return k(attn2, srcodd2)


def kernel(x, Wq, Wk, Wv, Wo):
    Bb, Tt, Dd = x.shape
    x_a = x[:, 0::2, :]
    x_b = x[:, 1::2, :]
    merged_full, tmask, src_odd = _match_pallas(x_a, x_b)
    w_cat = jnp.stack([Wq, Wk, Wv], axis=0).astype(jnp.bfloat16)
    qkv = _qkv_pallas(merged_full, w_cat)
    qkv_h = qkv.reshape(Bb, Tt, 3, _H, _HD).transpose(0, 2, 3, 1, 4)
    oh = _attn_pallas(qkv_h[:, 0], qkv_h[:, 1], qkv_h[:, 2],
                      tmask)
    o_cat = oh.transpose(0, 2, 1, 3).reshape(Bb, Tt, Dd)
    attn_out = _oproj_pallas(o_cat, Wo)
    out2 = _unmerge_sc(attn_out.reshape(Bb * Tt, Dd),
                       src_odd.reshape(Bb * _TA))
    return out2.reshape(Bb, Tt, Dd)
